# baseline (speedup 1.0000x reference)
import jax
import jax.numpy as jnp
from jax import lax
from jax.experimental import pallas as pl
from jax.experimental.pallas import tpu as pltpu
from jax.experimental.pallas import tpu_sc as plsc

_N = 10000
_E = 320000
_D = 128
_H = 128

_NC = 2
_NS = 16
_NW = _NC * _NS
_CH = 128
_CPW = 80
_NSTG = 2
_CPS = _CPW // _NSTG
_EPW = _CPW * _CH
_EPAD = _NW * _EPW
_NP = 10240
_NPT = _NP // _NS

_RB = 2000

_mesh = plsc.VectorSubcoreMesh(core_axis_name="c", subcore_axis_name="s",
                               num_cores=_NC, num_subcores=_NS)


def _make_sc_agg(compute_deg: bool):
    acc_ty = jax.ShapeDtypeStruct((_NC * _NP, _H), jnp.float32)
    out_type = ([acc_ty, jax.ShapeDtypeStruct((_NC * _NP,), jnp.float32)]
                if compute_deg else acc_ty)
    scratch = [
        pltpu.VMEM((_CH,), jnp.int32),
        pltpu.VMEM((_CH,), jnp.int32),
        pltpu.VMEM((_CH,), jnp.int32),
        pltpu.VMEM((_CH,), jnp.int32),
        pltpu.VMEM((_CH, _H), jnp.float32),
        pltpu.VMEM((_CH, _H), jnp.float32),
        pltpu.VMEM_SHARED((_NP, _H), jnp.float32),
        pltpu.SemaphoreType.DMA,
        pltpu.SemaphoreType.DMA,
    ]
    if compute_deg:
        scratch += [
            pltpu.VMEM((_CH,), jnp.float32),
            pltpu.VMEM((_NPT,), jnp.float32),
            pltpu.VMEM_SHARED((_NP,), jnp.float32),
        ]

    def body(src_hbm, dst_hbm, xt_hbm, z_hbm, *rest):
        if compute_deg:
            (acc_out, deg_out, src_ca, dst_ca, src_cb, dst_cb,
             rows_a, rows_b, acc, sem_a, sem_b, ones_c, dbuf, dega) = rest
        else:
            (acc_out, src_ca, dst_ca, src_cb, dst_cb,
             rows_a, rows_b, acc, sem_a, sem_b) = rest
        c = lax.axis_index("c")
        s = lax.axis_index("s")
        wid = s * _NC + c
        base = wid * _EPW

        for k in range(_NPT // _CH):
            r0 = s * _NPT + k * _CH
            pltpu.sync_copy(z_hbm.at[pl.ds(r0, _CH)], rows_a)
            pltpu.sync_copy(rows_a, acc.at[pl.ds(r0, _CH)])
        if compute_deg:
            for k in range(_NPT // 16):
                dbuf[pl.ds(16 * k, 16)] = jnp.zeros((16,), jnp.float32)
            for k in range(_CH // 16):
                ones_c[pl.ds(16 * k, 16)] = jnp.ones((16,), jnp.float32)
            pltpu.sync_copy(dbuf, dega.at[pl.ds(s * _NPT, _NPT)])
        plsc.subcore_barrier()

        def load_idx(j, src_c, dst_c):
            off = base + j * _CH
            pltpu.sync_copy(src_hbm.at[pl.ds(off, _CH)], src_c)
            pltpu.sync_copy(dst_hbm.at[pl.ds(off, _CH)], dst_c)

        def scatter(dst_c, rows):
            pltpu.sync_copy(rows, acc.at[dst_c], add=True)
            if compute_deg:
                pltpu.sync_copy(ones_c, dega.at[dst_c], add=True)

        @pl.loop(0, _CPW // 2)
        def _(g):
            j = 2 * g
            load_idx(j, src_ca, dst_ca)
            load_idx(j + 1, src_cb, dst_cb)
            da = pltpu.async_copy(xt_hbm.at[src_ca], rows_a, sem_a)
            db = pltpu.async_copy(xt_hbm.at[src_cb], rows_b, sem_b)
            da.wait()
            scatter(dst_ca, rows_a)
            db.wait()
            scatter(dst_cb, rows_b)

        plsc.subcore_barrier()
        for k in range(_NPT // _CH):
            r0 = s * _NPT + k * _CH
            pltpu.sync_copy(acc.at[pl.ds(r0, _CH)], rows_a)
            pltpu.sync_copy(rows_a, acc_out.at[pl.ds(c * _NP + r0, _CH)])
        if compute_deg:
            pltpu.sync_copy(dega.at[pl.ds(s * _NPT, _NPT)], dbuf)
            pltpu.sync_copy(dbuf, deg_out.at[pl.ds(c * _NP + s * _NPT, _NPT)])

    return pl.kernel(body, out_type=out_type, mesh=_mesh,
                     scratch_types=scratch)


_sc_agg_deg = _make_sc_agg(True)
_sc_agg = _make_sc_agg(False)


def _lin_body(x_ref, w_ref, b_ref, o_ref):
    o_ref[...] = (jnp.dot(x_ref[...], w_ref[...],
                          preferred_element_type=jnp.float32) + b_ref[...])


def _tc_linear(x, W, b):
    return pl.pallas_call(
        _lin_body,
        grid=(_N // _RB,),
        in_specs=[
            pl.BlockSpec((_RB, _D), lambda i: (i, 0)),
            pl.BlockSpec((_D, _H), lambda i: (0, 0)),
            pl.BlockSpec((1, _H), lambda i: (0, 0)),
        ],
        out_specs=pl.BlockSpec((_RB, _H), lambda i: (i, 0)),
        out_shape=jax.ShapeDtypeStruct((_N, _H), jnp.float32),
    )(x, W, b.reshape(1, _H))


def _norm_relu(m, g, be):
    mu = jnp.mean(m, axis=-1, keepdims=True)
    var = jnp.mean((m - mu) * (m - mu), axis=-1, keepdims=True)
    h = (m - mu) / jnp.sqrt(var + 1e-5) * g + be
    return jnp.maximum(h, 0.0)


def _mid_body(p0_ref, p1_ref, d0_ref, d1_ref, g_ref, be_ref, w_ref, b_ref,
              h_out, xt_out):
    deg = jnp.maximum(d0_ref[...] + d1_ref[...], 1.0)
    m = (p0_ref[0] + p1_ref[0]) / deg
    h = _norm_relu(m, g_ref[...], be_ref[...])
    h_out[...] = h
    xt_out[...] = (jnp.dot(h, w_ref[...],
                           preferred_element_type=jnp.float32) + b_ref[...])


def _tc_mid(acc, d0b, d1b, g, be, W, b):
    return pl.pallas_call(
        _mid_body,
        grid=(_N // _RB,),
        in_specs=[
            pl.BlockSpec((1, _RB, _H), lambda i: (0, i, 0)),
            pl.BlockSpec((1, _RB, _H), lambda i: (1, i, 0)),
            pl.BlockSpec((_RB, _H), lambda i: (i, 0)),
            pl.BlockSpec((_RB, _H), lambda i: (i, 0)),
            pl.BlockSpec((1, _H), lambda i: (0, 0)),
            pl.BlockSpec((1, _H), lambda i: (0, 0)),
            pl.BlockSpec((_H, _H), lambda i: (0, 0)),
            pl.BlockSpec((1, _H), lambda i: (0, 0)),
        ],
        out_specs=[
            pl.BlockSpec((_RB, _H), lambda i: (i, 0)),
            pl.BlockSpec((_RB, _H), lambda i: (i, 0)),
        ],
        out_shape=[
            jax.ShapeDtypeStruct((_N, _H), jnp.float32),
            jax.ShapeDtypeStruct((_N, _H), jnp.float32),
        ],
    )(acc, acc, d0b, d1b, g.reshape(1, _H), be.reshape(1, _H), W,
      b.reshape(1, _H))


def _fin_body(h1_ref, p0_ref, p1_ref, d0_ref, d1_ref, g_ref, be_ref,
              w_ref, b_ref, x0_ref, o_ref):
    deg = jnp.maximum(d0_ref[...] + d1_ref[...], 1.0)
    m = (p0_ref[0] + p1_ref[0]) / deg
    h = _norm_relu(h1_ref[...] + m, g_ref[...], be_ref[...])
    o_ref[...] = (jnp.dot(h, w_ref[...],
                          preferred_element_type=jnp.float32)
                  + b_ref[...] + x0_ref[...])


def _tc_final(h1, acc, d0b, d1b, g, be, Wout, bout, x0):
    return pl.pallas_call(
        _fin_body,
        grid=(_N // _RB,),
        in_specs=[
            pl.BlockSpec((_RB, _H), lambda i: (i, 0)),
            pl.BlockSpec((1, _RB, _H), lambda i: (0, i, 0)),
            pl.BlockSpec((1, _RB, _H), lambda i: (1, i, 0)),
            pl.BlockSpec((_RB, _H), lambda i: (i, 0)),
            pl.BlockSpec((_RB, _H), lambda i: (i, 0)),
            pl.BlockSpec((1, _H), lambda i: (0, 0)),
            pl.BlockSpec((1, _H), lambda i: (0, 0)),
            pl.BlockSpec((_H, _D), lambda i: (0, 0)),
            pl.BlockSpec((1, _D), lambda i: (0, 0)),
            pl.BlockSpec((_RB, _D), lambda i: (i, 0)),
        ],
        out_specs=pl.BlockSpec((_RB, _D), lambda i: (i, 0)),
        out_shape=jax.ShapeDtypeStruct((_N, _D), jnp.float32),
    )(h1, acc, acc, d0b, d1b, g.reshape(1, _H), be.reshape(1, _H),
      Wout, bout.reshape(1, _D), x0)


def kernel(subgraph_embeddings, edge_index, W0, b0, g0, be0,
           W1, b1, g1, be1, Wout, bout):
    x0 = subgraph_embeddings
    pad = _EPAD - _E
    srcp = jnp.concatenate([edge_index[0], jnp.zeros((pad,), jnp.int32)])
    dstp = jnp.concatenate([edge_index[1], jnp.full((pad,), _N, jnp.int32)])
    z128 = jnp.zeros((_NP, _H), jnp.float32)

    xt0 = _tc_linear(x0, W0, b0)
    acc0, degflat = _sc_agg_deg(srcp, dstp, xt0, z128)
    acc0 = acc0.reshape(_NC, _NP, _H)
    degp = degflat.reshape(_NC, _NP)
    d0b = jnp.broadcast_to(degp[0, :_N, None], (_N, _H))
    d1b = jnp.broadcast_to(degp[1, :_N, None], (_N, _H))
    h1, xt1 = _tc_mid(acc0, d0b, d1b, g0, be0, W1, b1)
    acc1 = _sc_agg(srcp, dstp, xt1, z128).reshape(_NC, _NP, _H)
    return _tc_final(h1, acc1, d0b, d1b, g1, be1, Wout, bout, x0)


# speedup vs baseline: 1.0871x; 1.0871x over previous
import jax
import jax.numpy as jnp
from jax import lax
from jax.experimental import pallas as pl
from jax.experimental.pallas import tpu as pltpu
from jax.experimental.pallas import tpu_sc as plsc

_N = 10000
_E = 320000
_D = 128
_H = 128

_NC = 2
_NS = 16
_NW = _NC * _NS
_CH = 128
_CPW = 80
_QPW = _CPW // 4
_EPW = _CPW * _CH
_EPAD = _NW * _EPW
_NP = 10240
_NPT = _NP // _NS

_RB = 2000

_mesh = plsc.VectorSubcoreMesh(core_axis_name="c", subcore_axis_name="s",
                               num_cores=_NC, num_subcores=_NS)


def _make_sc_agg(compute_deg: bool):
    acc_ty = jax.ShapeDtypeStruct((_NC * _NP, _H), jnp.float32)
    out_type = ([acc_ty, jax.ShapeDtypeStruct((_NC * _NP,), jnp.float32)]
                if compute_deg else acc_ty)
    scratch = [
        pltpu.VMEM((8, _CH), jnp.int32),
        pltpu.VMEM((_CH, _H), jnp.float32),
        pltpu.VMEM_SHARED((_NP, _H), jnp.float32),
        pltpu.SemaphoreType.DMA,
    ]
    if compute_deg:
        scratch += [
            pltpu.VMEM((_CH,), jnp.float32),
            pltpu.VMEM((_NPT,), jnp.float32),
            pltpu.VMEM_SHARED((_NP,), jnp.float32),
        ]

    def body(idx_hbm, xt_hbm, z_hbm, *rest):
        if compute_deg:
            (acc_out, deg_out, idx8, rows, acc, sem,
             ones_c, dbuf, dega) = rest
        else:
            acc_out, idx8, rows, acc, sem = rest
        c = lax.axis_index("c")
        s = lax.axis_index("s")
        wid = s * _NC + c

        for k in range(_NPT // _CH):
            r0 = s * _NPT + k * _CH
            pltpu.sync_copy(z_hbm.at[pl.ds(r0, _CH)], rows)
            pltpu.sync_copy(rows, acc.at[pl.ds(r0, _CH)])
        if compute_deg:
            for k in range(_NPT // 16):
                dbuf[pl.ds(16 * k, 16)] = jnp.zeros((16,), jnp.float32)
            for k in range(_CH // 16):
                ones_c[pl.ds(16 * k, 16)] = jnp.ones((16,), jnp.float32)
            pltpu.sync_copy(dbuf, dega.at[pl.ds(s * _NPT, _NPT)])
        plsc.subcore_barrier()

        @pl.loop(0, _QPW)
        def _(q):
            pltpu.sync_copy(idx_hbm.at[wid, q], idx8)
            for k in range(4):
                pltpu.async_copy(xt_hbm.at[idx8.at[k]], rows, sem).wait()
                pltpu.sync_copy(rows, acc.at[idx8.at[4 + k]], add=True)
                if compute_deg:
                    pltpu.sync_copy(ones_c, dega.at[idx8.at[4 + k]], add=True)

        plsc.subcore_barrier()
        for k in range(_NPT // _CH):
            r0 = s * _NPT + k * _CH
            pltpu.sync_copy(acc.at[pl.ds(r0, _CH)], rows)
            pltpu.sync_copy(rows, acc_out.at[pl.ds(c * _NP + r0, _CH)])
        if compute_deg:
            pltpu.sync_copy(dega.at[pl.ds(s * _NPT, _NPT)], dbuf)
            pltpu.sync_copy(dbuf, deg_out.at[pl.ds(c * _NP + s * _NPT, _NPT)])

    return pl.kernel(body, out_type=out_type, mesh=_mesh,
                     scratch_types=scratch)


_sc_agg_deg = _make_sc_agg(True)
_sc_agg = _make_sc_agg(False)


def _lin_body(x_ref, w_ref, b_ref, o_ref):
    o_ref[...] = (jnp.dot(x_ref[...], w_ref[...],
                          preferred_element_type=jnp.float32) + b_ref[...])


def _tc_linear(x, W, b):
    return pl.pallas_call(
        _lin_body,
        grid=(_N // _RB,),
        in_specs=[
            pl.BlockSpec((_RB, _D), lambda i: (i, 0)),
            pl.BlockSpec((_D, _H), lambda i: (0, 0)),
            pl.BlockSpec((1, _H), lambda i: (0, 0)),
        ],
        out_specs=pl.BlockSpec((_RB, _H), lambda i: (i, 0)),
        out_shape=jax.ShapeDtypeStruct((_N, _H), jnp.float32),
    )(x, W, b.reshape(1, _H))


def _norm_relu(m, g, be):
    mu = jnp.mean(m, axis=-1, keepdims=True)
    var = jnp.mean((m - mu) * (m - mu), axis=-1, keepdims=True)
    h = (m - mu) / jnp.sqrt(var + 1e-5) * g + be
    return jnp.maximum(h, 0.0)


def _mid_body(p0_ref, p1_ref, d0_ref, d1_ref, g_ref, be_ref, w_ref, b_ref,
              h_out, xt_out):
    deg = jnp.maximum(d0_ref[...] + d1_ref[...], 1.0)
    m = (p0_ref[0] + p1_ref[0]) / deg
    h = _norm_relu(m, g_ref[...], be_ref[...])
    h_out[...] = h
    xt_out[...] = (jnp.dot(h, w_ref[...],
                           preferred_element_type=jnp.float32) + b_ref[...])


def _tc_mid(acc, d0b, d1b, g, be, W, b):
    return pl.pallas_call(
        _mid_body,
        grid=(_N // _RB,),
        in_specs=[
            pl.BlockSpec((1, _RB, _H), lambda i: (0, i, 0)),
            pl.BlockSpec((1, _RB, _H), lambda i: (1, i, 0)),
            pl.BlockSpec((_RB, _H), lambda i: (i, 0)),
            pl.BlockSpec((_RB, _H), lambda i: (i, 0)),
            pl.BlockSpec((1, _H), lambda i: (0, 0)),
            pl.BlockSpec((1, _H), lambda i: (0, 0)),
            pl.BlockSpec((_H, _H), lambda i: (0, 0)),
            pl.BlockSpec((1, _H), lambda i: (0, 0)),
        ],
        out_specs=[
            pl.BlockSpec((_RB, _H), lambda i: (i, 0)),
            pl.BlockSpec((_RB, _H), lambda i: (i, 0)),
        ],
        out_shape=[
            jax.ShapeDtypeStruct((_N, _H), jnp.float32),
            jax.ShapeDtypeStruct((_N, _H), jnp.float32),
        ],
    )(acc, acc, d0b, d1b, g.reshape(1, _H), be.reshape(1, _H), W,
      b.reshape(1, _H))


def _fin_body(h1_ref, p0_ref, p1_ref, d0_ref, d1_ref, g_ref, be_ref,
              w_ref, b_ref, x0_ref, o_ref):
    deg = jnp.maximum(d0_ref[...] + d1_ref[...], 1.0)
    m = (p0_ref[0] + p1_ref[0]) / deg
    h = _norm_relu(h1_ref[...] + m, g_ref[...], be_ref[...])
    o_ref[...] = (jnp.dot(h, w_ref[...],
                          preferred_element_type=jnp.float32)
                  + b_ref[...] + x0_ref[...])


def _tc_final(h1, acc, d0b, d1b, g, be, Wout, bout, x0):
    return pl.pallas_call(
        _fin_body,
        grid=(_N // _RB,),
        in_specs=[
            pl.BlockSpec((_RB, _H), lambda i: (i, 0)),
            pl.BlockSpec((1, _RB, _H), lambda i: (0, i, 0)),
            pl.BlockSpec((1, _RB, _H), lambda i: (1, i, 0)),
            pl.BlockSpec((_RB, _H), lambda i: (i, 0)),
            pl.BlockSpec((_RB, _H), lambda i: (i, 0)),
            pl.BlockSpec((1, _H), lambda i: (0, 0)),
            pl.BlockSpec((1, _H), lambda i: (0, 0)),
            pl.BlockSpec((_H, _D), lambda i: (0, 0)),
            pl.BlockSpec((1, _D), lambda i: (0, 0)),
            pl.BlockSpec((_RB, _D), lambda i: (i, 0)),
        ],
        out_specs=pl.BlockSpec((_RB, _D), lambda i: (i, 0)),
        out_shape=jax.ShapeDtypeStruct((_N, _D), jnp.float32),
    )(h1, acc, acc, d0b, d1b, g.reshape(1, _H), be.reshape(1, _H),
      Wout, bout.reshape(1, _D), x0)


def kernel(subgraph_embeddings, edge_index, W0, b0, g0, be0,
           W1, b1, g1, be1, Wout, bout):
    x0 = subgraph_embeddings
    pad = _EPAD - _E
    srcp = jnp.concatenate(
        [edge_index[0], jnp.zeros((pad,), jnp.int32)]
    ).reshape(_NW, _QPW, 4, _CH)
    dstp = jnp.concatenate(
        [edge_index[1], jnp.full((pad,), _N, jnp.int32)]
    ).reshape(_NW, _QPW, 4, _CH)
    idxp = jnp.concatenate([srcp, dstp], axis=2)
    z128 = jnp.zeros((_NP, _H), jnp.float32)

    xt0 = _tc_linear(x0, W0, b0)
    acc0, degflat = _sc_agg_deg(idxp, xt0, z128)
    acc0 = acc0.reshape(_NC, _NP, _H)
    degp = degflat.reshape(_NC, _NP)
    d0b = jnp.broadcast_to(degp[0, :_N, None], (_N, _H))
    d1b = jnp.broadcast_to(degp[1, :_N, None], (_N, _H))
    h1, xt1 = _tc_mid(acc0, d0b, d1b, g0, be0, W1, b1)
    acc1 = _sc_agg(idxp, xt1, z128).reshape(_NC, _NP, _H)
    return _tc_final(h1, acc1, d0b, d1b, g1, be1, Wout, bout, x0)


# speedup vs baseline: 1.2899x; 1.1866x over previous
import jax
import jax.numpy as jnp
from jax import lax
from jax.experimental import pallas as pl
from jax.experimental.pallas import tpu as pltpu
from jax.experimental.pallas import tpu_sc as plsc

_N = 10000
_E = 320000
_D = 128
_H = 128

_NC = 2
_NS = 16
_NW = _NC * _NS
_CH = 128
_CPP = 158
_CA = 63
_CB = _CPP - _CA
_EPAD = _NS * _CPP * _CH
_NP = 10240
_NPT = _NP // _NS

_RB = 2000

_mesh = plsc.VectorSubcoreMesh(core_axis_name="c", subcore_axis_name="s",
                               num_cores=_NC, num_subcores=_NS)


def _make_sc_agg(compute_deg: bool):
    acc_ty = jax.ShapeDtypeStruct((_NC * _NP, _H), jnp.float32)
    out_type = ([acc_ty, jax.ShapeDtypeStruct((_NC * _NP,), jnp.float32)]
                if compute_deg else acc_ty)
    scratch = [
        pltpu.VMEM((_CH,), jnp.int32),
        pltpu.VMEM((_CH,), jnp.int32),
        pltpu.VMEM((_CH, _H), jnp.float32),
        pltpu.VMEM_SHARED((_NP, _H), jnp.float32),
        pltpu.SemaphoreType.DMA,
    ]
    if compute_deg:
        scratch += [
            pltpu.VMEM((_CH,), jnp.float32),
            pltpu.VMEM((_NPT,), jnp.float32),
            pltpu.VMEM_SHARED((_NP,), jnp.float32),
        ]

    def body(src_hbm, dst_hbm, xt_hbm, z_hbm, *rest):
        if compute_deg:
            (acc_out, deg_out, src_c, dst_c, rows, acc, sem,
             ones_c, dbuf, dega) = rest
        else:
            acc_out, src_c, dst_c, rows, acc, sem = rest
        c = lax.axis_index("c")
        s = lax.axis_index("s")
        base = (s * _CPP + c * _CA) * _CH
        n_chunks = jnp.where(c == 0, _CA, _CB)

        for k in range(_NPT // _CH):
            r0 = s * _NPT + k * _CH
            pltpu.sync_copy(z_hbm.at[pl.ds(r0, _CH)], rows)
            pltpu.sync_copy(rows, acc.at[pl.ds(r0, _CH)])
        if compute_deg:
            for k in range(_NPT // 16):
                dbuf[pl.ds(16 * k, 16)] = jnp.zeros((16,), jnp.float32)
            for k in range(_CH // 16):
                ones_c[pl.ds(16 * k, 16)] = jnp.ones((16,), jnp.float32)
            pltpu.sync_copy(dbuf, dega.at[pl.ds(s * _NPT, _NPT)])
        plsc.subcore_barrier()

        @pl.loop(0, n_chunks)
        def _(j):
            off = base + j * _CH
            pltpu.sync_copy(src_hbm.at[pl.ds(off, _CH)], src_c)
            pltpu.sync_copy(dst_hbm.at[pl.ds(off, _CH)], dst_c)
            pltpu.async_copy(xt_hbm.at[src_c], rows, sem).wait()
            pltpu.sync_copy(rows, acc.at[dst_c], add=True)
            if compute_deg:
                pltpu.sync_copy(ones_c, dega.at[dst_c], add=True)

        plsc.subcore_barrier()
        for k in range(_NPT // _CH):
            r0 = s * _NPT + k * _CH
            pltpu.sync_copy(acc.at[pl.ds(r0, _CH)], rows)
            pltpu.sync_copy(rows, acc_out.at[pl.ds(c * _NP + r0, _CH)])
        if compute_deg:
            pltpu.sync_copy(dega.at[pl.ds(s * _NPT, _NPT)], dbuf)
            pltpu.sync_copy(dbuf, deg_out.at[pl.ds(c * _NP + s * _NPT, _NPT)])

    return pl.kernel(body, out_type=out_type, mesh=_mesh,
                     scratch_types=scratch)


_sc_agg_deg = _make_sc_agg(True)
_sc_agg = _make_sc_agg(False)


def _lin_body(x_ref, w_ref, b_ref, o_ref):
    o_ref[...] = (jnp.dot(x_ref[...], w_ref[...],
                          preferred_element_type=jnp.float32) + b_ref[...])


def _tc_linear(x, W, b):
    return pl.pallas_call(
        _lin_body,
        grid=(_N // _RB,),
        in_specs=[
            pl.BlockSpec((_RB, _D), lambda i: (i, 0)),
            pl.BlockSpec((_D, _H), lambda i: (0, 0)),
            pl.BlockSpec((1, _H), lambda i: (0, 0)),
        ],
        out_specs=pl.BlockSpec((_RB, _H), lambda i: (i, 0)),
        out_shape=jax.ShapeDtypeStruct((_N, _H), jnp.float32),
    )(x, W, b.reshape(1, _H))


def _norm_relu(m, g, be):
    mu = jnp.mean(m, axis=-1, keepdims=True)
    var = jnp.mean((m - mu) * (m - mu), axis=-1, keepdims=True)
    h = (m - mu) / jnp.sqrt(var + 1e-5) * g + be
    return jnp.maximum(h, 0.0)


def _mid_body(p0_ref, p1_ref, d0_ref, d1_ref, g_ref, be_ref, w_ref, b_ref,
              h_out, xt_out):
    deg = jnp.maximum(d0_ref[...] + d1_ref[...], 1.0)
    m = (p0_ref[0] + p1_ref[0]) / deg
    h = _norm_relu(m, g_ref[...], be_ref[...])
    h_out[...] = h
    xt_out[...] = (jnp.dot(h, w_ref[...],
                           preferred_element_type=jnp.float32) + b_ref[...])


def _tc_mid(acc, d0b, d1b, g, be, W, b):
    return pl.pallas_call(
        _mid_body,
        grid=(_N // _RB,),
        in_specs=[
            pl.BlockSpec((1, _RB, _H), lambda i: (0, i, 0)),
            pl.BlockSpec((1, _RB, _H), lambda i: (1, i, 0)),
            pl.BlockSpec((_RB, _H), lambda i: (i, 0)),
            pl.BlockSpec((_RB, _H), lambda i: (i, 0)),
            pl.BlockSpec((1, _H), lambda i: (0, 0)),
            pl.BlockSpec((1, _H), lambda i: (0, 0)),
            pl.BlockSpec((_H, _H), lambda i: (0, 0)),
            pl.BlockSpec((1, _H), lambda i: (0, 0)),
        ],
        out_specs=[
            pl.BlockSpec((_RB, _H), lambda i: (i, 0)),
            pl.BlockSpec((_RB, _H), lambda i: (i, 0)),
        ],
        out_shape=[
            jax.ShapeDtypeStruct((_N, _H), jnp.float32),
            jax.ShapeDtypeStruct((_N, _H), jnp.float32),
        ],
    )(acc, acc, d0b, d1b, g.reshape(1, _H), be.reshape(1, _H), W,
      b.reshape(1, _H))


def _fin_body(h1_ref, p0_ref, p1_ref, d0_ref, d1_ref, g_ref, be_ref,
              w_ref, b_ref, x0_ref, o_ref):
    deg = jnp.maximum(d0_ref[...] + d1_ref[...], 1.0)
    m = (p0_ref[0] + p1_ref[0]) / deg
    h = _norm_relu(h1_ref[...] + m, g_ref[...], be_ref[...])
    o_ref[...] = (jnp.dot(h, w_ref[...],
                          preferred_element_type=jnp.float32)
                  + b_ref[...] + x0_ref[...])


def _tc_final(h1, acc, d0b, d1b, g, be, Wout, bout, x0):
    return pl.pallas_call(
        _fin_body,
        grid=(_N // _RB,),
        in_specs=[
            pl.BlockSpec((_RB, _H), lambda i: (i, 0)),
            pl.BlockSpec((1, _RB, _H), lambda i: (0, i, 0)),
            pl.BlockSpec((1, _RB, _H), lambda i: (1, i, 0)),
            pl.BlockSpec((_RB, _H), lambda i: (i, 0)),
            pl.BlockSpec((_RB, _H), lambda i: (i, 0)),
            pl.BlockSpec((1, _H), lambda i: (0, 0)),
            pl.BlockSpec((1, _H), lambda i: (0, 0)),
            pl.BlockSpec((_H, _D), lambda i: (0, 0)),
            pl.BlockSpec((1, _D), lambda i: (0, 0)),
            pl.BlockSpec((_RB, _D), lambda i: (i, 0)),
        ],
        out_specs=pl.BlockSpec((_RB, _D), lambda i: (i, 0)),
        out_shape=jax.ShapeDtypeStruct((_N, _D), jnp.float32),
    )(h1, acc, acc, d0b, d1b, g.reshape(1, _H), be.reshape(1, _H),
      Wout, bout.reshape(1, _D), x0)


def kernel(subgraph_embeddings, edge_index, W0, b0, g0, be0,
           W1, b1, g1, be1, Wout, bout):
    x0 = subgraph_embeddings
    pad = _EPAD - _E
    srcp = jnp.concatenate([edge_index[0], jnp.zeros((pad,), jnp.int32)])
    dstp = jnp.concatenate([edge_index[1], jnp.full((pad,), _N, jnp.int32)])
    z128 = jnp.zeros((_NP, _H), jnp.float32)

    xt0 = _tc_linear(x0, W0, b0)
    acc0, degflat = _sc_agg_deg(srcp, dstp, xt0, z128)
    acc0 = acc0.reshape(_NC, _NP, _H)
    degp = degflat.reshape(_NC, _NP)
    d0b = jnp.broadcast_to(degp[0, :_N, None], (_N, _H))
    d1b = jnp.broadcast_to(degp[1, :_N, None], (_N, _H))
    h1, xt1 = _tc_mid(acc0, d0b, d1b, g0, be0, W1, b1)
    acc1 = _sc_agg(srcp, dstp, xt1, z128).reshape(_NC, _NP, _H)
    return _tc_final(h1, acc1, d0b, d1b, g1, be1, Wout, bout, x0)


# speedup vs baseline: 1.5645x; 1.2129x over previous
import jax
import jax.numpy as jnp
from jax import lax
from jax.experimental import pallas as pl
from jax.experimental.pallas import tpu as pltpu
from jax.experimental.pallas import tpu_sc as plsc

_N = 10000
_E = 320000
_D = 128
_H = 128

_NC = 2
_NS = 16
_NW = _NC * _NS
_CH = 128
_CPP = 158
_CA = 95
_CB = _CPP - _CA
_EPAD = _NS * _CPP * _CH
_NP = 10240
_NPT = _NP // _NS

_RB = 2000

_mesh = plsc.VectorSubcoreMesh(core_axis_name="c", subcore_axis_name="s",
                               num_cores=_NC, num_subcores=_NS)


def _make_sc_agg(compute_deg: bool):
    acc_ty = jax.ShapeDtypeStruct((_NC * _NP, _H), jnp.float32)
    out_type = ([acc_ty, jax.ShapeDtypeStruct((_NC * _NP,), jnp.float32)]
                if compute_deg else acc_ty)
    scratch = [
        pltpu.VMEM((_CH,), jnp.int32),
        pltpu.VMEM((_CH,), jnp.int32),
        pltpu.VMEM((_CH, _H), jnp.float32),
        pltpu.VMEM_SHARED((_NP, _H), jnp.float32),
        pltpu.SemaphoreType.DMA,
    ]
    if compute_deg:
        scratch += [
            pltpu.VMEM((_CH,), jnp.float32),
            pltpu.VMEM((_NPT,), jnp.float32),
            pltpu.VMEM_SHARED((_NP,), jnp.float32),
        ]

    def body(src_hbm, dst_hbm, xt_hbm, z_hbm, *rest):
        if compute_deg:
            (acc_out, deg_out, src_c, dst_c, rows, acc, sem,
             ones_c, dbuf, dega) = rest
        else:
            acc_out, src_c, dst_c, rows, acc, sem = rest
        c = lax.axis_index("c")
        s = lax.axis_index("s")
        base = (s * _CPP + c * _CA) * _CH
        n_chunks = jnp.where(c == 0, _CA, _CB)

        for k in range(_NPT // _CH):
            r0 = s * _NPT + k * _CH
            pltpu.sync_copy(z_hbm.at[pl.ds(r0, _CH)], rows)
            pltpu.sync_copy(rows, acc.at[pl.ds(r0, _CH)])
        if compute_deg:
            for k in range(_NPT // 16):
                dbuf[pl.ds(16 * k, 16)] = jnp.zeros((16,), jnp.float32)
            for k in range(_CH // 16):
                ones_c[pl.ds(16 * k, 16)] = jnp.ones((16,), jnp.float32)
            pltpu.sync_copy(dbuf, dega.at[pl.ds(s * _NPT, _NPT)])
        plsc.subcore_barrier()

        @pl.loop(0, n_chunks)
        def _(j):
            off = base + j * _CH
            pltpu.sync_copy(src_hbm.at[pl.ds(off, _CH)], src_c)
            pltpu.sync_copy(dst_hbm.at[pl.ds(off, _CH)], dst_c)
            pltpu.async_copy(xt_hbm.at[src_c], rows, sem).wait()
            pltpu.sync_copy(rows, acc.at[dst_c], add=True)
            if compute_deg:
                pltpu.sync_copy(ones_c, dega.at[dst_c], add=True)

        plsc.subcore_barrier()
        for k in range(_NPT // _CH):
            r0 = s * _NPT + k * _CH
            pltpu.sync_copy(acc.at[pl.ds(r0, _CH)], rows)
            pltpu.sync_copy(rows, acc_out.at[pl.ds(c * _NP + r0, _CH)])
        if compute_deg:
            pltpu.sync_copy(dega.at[pl.ds(s * _NPT, _NPT)], dbuf)
            pltpu.sync_copy(dbuf, deg_out.at[pl.ds(c * _NP + s * _NPT, _NPT)])

    return pl.kernel(body, out_type=out_type, mesh=_mesh,
                     scratch_types=scratch)


_sc_agg_deg = _make_sc_agg(True)
_sc_agg = _make_sc_agg(False)


def _lin_body(x_ref, w_ref, b_ref, o_ref):
    o_ref[...] = (jnp.dot(x_ref[...], w_ref[...],
                          preferred_element_type=jnp.float32) + b_ref[...])


def _tc_linear(x, W, b):
    return pl.pallas_call(
        _lin_body,
        grid=(_N // _RB,),
        in_specs=[
            pl.BlockSpec((_RB, _D), lambda i: (i, 0)),
            pl.BlockSpec((_D, _H), lambda i: (0, 0)),
            pl.BlockSpec((1, _H), lambda i: (0, 0)),
        ],
        out_specs=pl.BlockSpec((_RB, _H), lambda i: (i, 0)),
        out_shape=jax.ShapeDtypeStruct((_N, _H), jnp.float32),
    )(x, W, b.reshape(1, _H))


def _norm_relu(m, g, be):
    mu = jnp.mean(m, axis=-1, keepdims=True)
    var = jnp.mean((m - mu) * (m - mu), axis=-1, keepdims=True)
    h = (m - mu) / jnp.sqrt(var + 1e-5) * g + be
    return jnp.maximum(h, 0.0)


def _mid_body(p0_ref, p1_ref, d0_ref, d1_ref, g_ref, be_ref, w_ref, b_ref,
              h_out, xt_out):
    deg = jnp.maximum(d0_ref[...] + d1_ref[...], 1.0)
    m = (p0_ref[0] + p1_ref[0]) / deg
    h = _norm_relu(m, g_ref[...], be_ref[...])
    h_out[...] = h
    xt_out[...] = (jnp.dot(h, w_ref[...],
                           preferred_element_type=jnp.float32) + b_ref[...])


def _tc_mid(acc, d0b, d1b, g, be, W, b):
    return pl.pallas_call(
        _mid_body,
        grid=(_N // _RB,),
        in_specs=[
            pl.BlockSpec((1, _RB, _H), lambda i: (0, i, 0)),
            pl.BlockSpec((1, _RB, _H), lambda i: (1, i, 0)),
            pl.BlockSpec((_RB, _H), lambda i: (i, 0)),
            pl.BlockSpec((_RB, _H), lambda i: (i, 0)),
            pl.BlockSpec((1, _H), lambda i: (0, 0)),
            pl.BlockSpec((1, _H), lambda i: (0, 0)),
            pl.BlockSpec((_H, _H), lambda i: (0, 0)),
            pl.BlockSpec((1, _H), lambda i: (0, 0)),
        ],
        out_specs=[
            pl.BlockSpec((_RB, _H), lambda i: (i, 0)),
            pl.BlockSpec((_RB, _H), lambda i: (i, 0)),
        ],
        out_shape=[
            jax.ShapeDtypeStruct((_N, _H), jnp.float32),
            jax.ShapeDtypeStruct((_N, _H), jnp.float32),
        ],
    )(acc, acc, d0b, d1b, g.reshape(1, _H), be.reshape(1, _H), W,
      b.reshape(1, _H))


def _fin_body(h1_ref, p0_ref, p1_ref, d0_ref, d1_ref, g_ref, be_ref,
              w_ref, b_ref, x0_ref, o_ref):
    deg = jnp.maximum(d0_ref[...] + d1_ref[...], 1.0)
    m = (p0_ref[0] + p1_ref[0]) / deg
    h = _norm_relu(h1_ref[...] + m, g_ref[...], be_ref[...])
    o_ref[...] = (jnp.dot(h, w_ref[...],
                          preferred_element_type=jnp.float32)
                  + b_ref[...] + x0_ref[...])


def _tc_final(h1, acc, d0b, d1b, g, be, Wout, bout, x0):
    return pl.pallas_call(
        _fin_body,
        grid=(_N // _RB,),
        in_specs=[
            pl.BlockSpec((_RB, _H), lambda i: (i, 0)),
            pl.BlockSpec((1, _RB, _H), lambda i: (0, i, 0)),
            pl.BlockSpec((1, _RB, _H), lambda i: (1, i, 0)),
            pl.BlockSpec((_RB, _H), lambda i: (i, 0)),
            pl.BlockSpec((_RB, _H), lambda i: (i, 0)),
            pl.BlockSpec((1, _H), lambda i: (0, 0)),
            pl.BlockSpec((1, _H), lambda i: (0, 0)),
            pl.BlockSpec((_H, _D), lambda i: (0, 0)),
            pl.BlockSpec((1, _D), lambda i: (0, 0)),
            pl.BlockSpec((_RB, _D), lambda i: (i, 0)),
        ],
        out_specs=pl.BlockSpec((_RB, _D), lambda i: (i, 0)),
        out_shape=jax.ShapeDtypeStruct((_N, _D), jnp.float32),
    )(h1, acc, acc, d0b, d1b, g.reshape(1, _H), be.reshape(1, _H),
      Wout, bout.reshape(1, _D), x0)


def kernel(subgraph_embeddings, edge_index, W0, b0, g0, be0,
           W1, b1, g1, be1, Wout, bout):
    x0 = subgraph_embeddings
    pad = _EPAD - _E
    srcp = jnp.concatenate([edge_index[0], jnp.zeros((pad,), jnp.int32)])
    dstp = jnp.concatenate([edge_index[1], jnp.full((pad,), _N, jnp.int32)])
    z128 = jnp.zeros((_NP, _H), jnp.float32)

    xt0 = _tc_linear(x0, W0, b0)
    acc0, degflat = _sc_agg_deg(srcp, dstp, xt0, z128)
    acc0 = acc0.reshape(_NC, _NP, _H)
    degp = degflat.reshape(_NC, _NP)
    d0b = jnp.broadcast_to(degp[0, :_N, None], (_N, _H))
    d1b = jnp.broadcast_to(degp[1, :_N, None], (_N, _H))
    h1, xt1 = _tc_mid(acc0, d0b, d1b, g0, be0, W1, b1)
    acc1 = _sc_agg(srcp, dstp, xt1, z128).reshape(_NC, _NP, _H)
    return _tc_final(h1, acc1, d0b, d1b, g1, be1, Wout, bout, x0)
